# baseline (speedup 1.0000x reference)
import functools
import jax
import jax.numpy as jnp
from jax import lax
from jax.experimental import pallas as pl
from jax.experimental.pallas import tpu as pltpu
from jax.experimental.pallas import tpu_sc as plsc

NC = 2
NS = 16
LN = 16
DH = 32

def _mesh():
    return plsc.VectorSubcoreMesh(core_axis_name="c", subcore_axis_name="s",
                                  num_cores=NC, num_subcores=NS)


def _make_spmm(NP, E_pad, C, S):
    J = C // S
    ET = E_pad // NS
    NCH = ET // C
    RPT = NP // NS
    ZR = 184
    RING = 3
    GPC = C // LN
    GPS = S // LN

    @functools.partial(
        pl.kernel,
        mesh=_mesh(),
        out_type=jax.ShapeDtypeStruct((NP, NC * DH), jnp.float32),
        scratch_types=[
            pltpu.VMEM((RING, 2, J, S), jnp.int32),
            pltpu.VMEM((RING, C), jnp.float32),
            pltpu.VMEM((RING, J, S), jnp.int32),
            pltpu.VMEM((RING, C, DH), jnp.float32),
            pltpu.VMEM_SHARED((NP, DH), jnp.float32),
            pltpu.SemaphoreType.DMA,
            pltpu.SemaphoreType.DMA,
            pltpu.SemaphoreType.DMA,
        ],
        compiler_params=pltpu.CompilerParams(use_tc_tiling_on_sc=False),
    )
    def spmm(ego2, edat, vals, side, ebuf, valv, rowS, rows_v,
             acc, lsem, gsem, ssem):
        c = lax.axis_index("c")
        s = lax.axis_index("s")
        zv = jnp.zeros((LN,), jnp.float32)

        def zfill(i, _):
            rows_v[0, i, pl.ds(0, LN)] = zv
            rows_v[0, i, pl.ds(LN, LN)] = zv
            return _
        lax.fori_loop(0, ZR, zfill, None)
        def zcopy(j, _):
            pltpu.sync_copy(rows_v.at[0, pl.ds(0, ZR)],
                            acc.at[pl.ds(s * RPT + j * ZR, ZR)])
            return _
        lax.fori_loop(0, RPT // ZR, zcopy, None)
        plsc.subcore_barrier()

        cb0 = s * NCH

        def issue_linear(k, r):
            pltpu.async_copy(edat.at[cb0 + k], ebuf.at[r], lsem)
            pltpu.async_copy(vals.at[pl.ds((cb0 + k) * C, C)], valv.at[r],
                             lsem)

        def wait_linear(r):
            pltpu.make_async_copy(edat.at[0], ebuf.at[r], lsem).wait()
            pltpu.make_async_copy(vals.at[pl.ds(0, C)], valv.at[r],
                                  lsem).wait()

        def idx_compute(r):
            def ib(i, _):
                j = i // GPS
                g = i % GPS
                v = ebuf[r, 0, j, pl.ds(g * LN, LN)]
                ebuf[r, 0, j, pl.ds(g * LN, LN)] = v * 2 + c
                return _
            lax.fori_loop(0, GPC, ib, None)

        def issue_gather(r):
            for j in range(J):
                pltpu.async_copy(ego2.at[ebuf.at[r, 0, j]],
                                 rows_v.at[r, pl.ds(j * S, S)], gsem)

        def wait_gather(r):
            for j in range(J):
                pltpu.make_async_copy(ego2.at[ebuf.at[r, 0, j]],
                                      rows_v.at[r, pl.ds(j * S, S)],
                                      gsem).wait()

        def scale(r):
            def sg(g, _):
                j = g // GPS
                gg = g % GPS
                rowS[r, j, pl.ds(gg * LN, LN)] = ebuf[r, 1, j, pl.ds(gg * LN, LN)]
                vv = valv[r, pl.ds(g * LN, LN)]
                for i in range(LN):
                    sval = vv.at[jnp.full((LN,), i, jnp.int32)].get(
                        mode='promise_in_bounds')
                    e = g * LN + i
                    rows_v[r, e, pl.ds(0, LN)] = rows_v[r, e, pl.ds(0, LN)] * sval
                    rows_v[r, e, pl.ds(LN, LN)] = rows_v[r, e, pl.ds(LN, LN)] * sval
                return _
            lax.fori_loop(0, GPC, sg, None)

        def issue_scatter(r):
            for j in range(J):
                pltpu.async_copy(rows_v.at[r, pl.ds(j * S, S)],
                                 acc.at[rowS.at[r, j]], ssem, add=True)

        def wait_scatter(r):
            for j in range(J):
                pltpu.make_async_copy(rows_v.at[r, pl.ds(j * S, S)],
                                      acc.at[rowS.at[r, j]], ssem).wait()

        issue_linear(0, 0)
        issue_linear(1, 1)
        issue_linear(2, 2)
        wait_linear(0)
        idx_compute(0)
        issue_gather(0)

        def step(k, r):
            r1 = (r + 1) % RING

            @pl.when(k + 1 < NCH)
            def _():
                wait_linear(r1)
                idx_compute(r1)

            @pl.when(jnp.logical_and(k + 1 < NCH, k >= 2))
            def _():
                wait_scatter(r1)

            @pl.when(k + 1 < NCH)
            def _():
                issue_gather(r1)

            wait_gather(r)
            scale(r)
            issue_scatter(r)

            @pl.when(k + RING < NCH)
            def _():
                issue_linear(k + RING, r)

        def outer(m, _):
            for u in range(RING):
                step(m * RING + u, u)
            return _
        lax.fori_loop(0, NCH // RING, outer, None)

        for u in range(RING):
            wait_scatter(u)

        plsc.subcore_barrier()
        pltpu.sync_copy(acc.at[pl.ds(s * RPT, RPT)],
                        side.at[pl.ds(s * RPT, RPT), pl.ds(c * DH, DH)])

    return spmm


def _dense_body(side_ref, ego_ref, wg_ref, bg_ref, wb_ref, bb_ref,
                h_ref, n_ref):
    side = side_ref[...]
    ego = ego_ref[...]
    su = jnp.dot(side, wg_ref[...], preferred_element_type=jnp.float32,
                 precision=lax.Precision.HIGHEST)
    bi = jnp.dot(ego * side, wb_ref[...], preferred_element_type=jnp.float32,
                 precision=lax.Precision.HIGHEST)
    x = su + bg_ref[...] + bi + bb_ref[...]
    h = jnp.where(x >= 0, x, 0.2 * x)
    h_ref[...] = h
    nrm = jnp.sqrt(jnp.sum(h * h, axis=1, keepdims=True))
    n_ref[...] = h / jnp.maximum(nrm, 1e-12)


def _dense(ego, side, wg, bg, wb, bb, blk=2176):
    N, D = ego.shape
    grid = N // blk
    return pl.pallas_call(
        _dense_body,
        grid=(grid,),
        in_specs=[
            pl.BlockSpec((blk, D), lambda i: (i, 0)),
            pl.BlockSpec((blk, D), lambda i: (i, 0)),
            pl.BlockSpec((D, D), lambda i: (0, 0)),
            pl.BlockSpec((1, D), lambda i: (0, 0)),
            pl.BlockSpec((D, D), lambda i: (0, 0)),
            pl.BlockSpec((1, D), lambda i: (0, 0)),
        ],
        out_specs=[pl.BlockSpec((blk, D), lambda i: (i, 0)),
                   pl.BlockSpec((blk, D), lambda i: (i, 0))],
        out_shape=[jax.ShapeDtypeStruct((N, D), jnp.float32),
                   jax.ShapeDtypeStruct((N, D), jnp.float32)],
    )(side, ego, wg, bg, wb, bb)


def _make_gather_out(N_user, B, D):
    BPW = B // (NC * NS)

    @functools.partial(
        pl.kernel,
        mesh=_mesh(),
        out_type=jax.ShapeDtypeStruct((2, 3, B, D), jnp.float32),
        scratch_types=[
            pltpu.VMEM((BPW,), jnp.int32),
            pltpu.VMEM((BPW, D), jnp.float32),
            pltpu.SemaphoreType.DMA,
        ],
        compiler_params=pltpu.CompilerParams(use_tc_tiling_on_sc=False),
    )
    def gather_out(ego0, norm1, norm2, users, items, out, idxv, rows_v, sem):
        c = lax.axis_index("c")
        s = lax.axis_index("s")
        w = s * NC + c
        tabs = [ego0, norm1, norm2]
        for ui, src in enumerate([users, items]):
            pltpu.sync_copy(src.at[pl.ds(w * BPW, BPW)], idxv)
            if ui == 1:
                def off(g, _):
                    idxv[pl.ds(g * LN, LN)] = idxv[pl.ds(g * LN, LN)] + N_user
                    return _
                lax.fori_loop(0, BPW // LN, off, None)
            for t in range(3):
                pltpu.async_copy(tabs[t].at[idxv], rows_v, sem).wait()
                pltpu.sync_copy(rows_v, out.at[ui, t, pl.ds(w * BPW, BPW)])

    return gather_out


def kernel(users, items, adj_indices, adj_vals, user_emb, item_emb,
           W_gc_0, b_gc_0, W_bi_0, b_bi_0, W_gc_1, b_gc_1, W_bi_1, b_bi_1):
    N_user, D = user_emb.shape
    N = N_user + item_emb.shape[0]
    E = adj_vals.shape[0]
    B = users.shape[0]

    C, S = 256, 128
    EQ = NS * C * 3
    E_pad = ((E + EQ - 1) // EQ) * EQ
    pad = E_pad - E
    NP = ((N + NS * 8 - 1) // (NS * 8)) * (NS * 8)

    row = adj_indices[0]
    col = adj_indices[1]
    if pad:
        z = jnp.zeros((pad,), jnp.int32)
        row = jnp.concatenate([row, z])
        col = jnp.concatenate([col, z])
        vals = jnp.concatenate([adj_vals, jnp.zeros((pad,), jnp.float32)])
    else:
        vals = adj_vals
    J = C // S
    edat = jnp.stack([col.reshape(-1, J, S),
                      row.reshape(-1, J, S)], axis=1)

    spmm = _make_spmm(NP, E_pad, C, S)
    gather_out = _make_gather_out(N_user, B, D)

    ego0 = jnp.concatenate(
        [user_emb, item_emb,
         jnp.zeros((NP - N, D), jnp.float32)], axis=0)

    side1 = spmm(ego0.reshape(2 * NP, DH), edat, vals)
    h1, norm1 = _dense(ego0, side1, W_gc_0, b_gc_0, W_bi_0, b_bi_0)

    side2 = spmm(h1.reshape(2 * NP, DH), edat, vals)
    h2, norm2 = _dense(h1, side2, W_gc_1, b_gc_1, W_bi_1, b_bi_1)

    outg = gather_out(ego0, norm1, norm2, users, items)
    u_g = outg[0].transpose(1, 0, 2).reshape(B, 3 * D)
    i_g = outg[1].transpose(1, 0, 2).reshape(B, 3 * D)
    return (u_g, i_g)


# speedup vs baseline: 1.2857x; 1.2857x over previous
import functools
import jax
import jax.numpy as jnp
from jax import lax
from jax.experimental import pallas as pl
from jax.experimental.pallas import tpu as pltpu
from jax.experimental.pallas import tpu_sc as plsc

NC = 2
NS = 16
LN = 16
DH = 32

def _mesh():
    return plsc.VectorSubcoreMesh(core_axis_name="c", subcore_axis_name="s",
                                  num_cores=NC, num_subcores=NS)


def _make_spmm(NP, E_pad, C, S):
    J = C // S
    ET = E_pad // NS
    NCH = ET // C
    RPT = NP // NS
    ZR = 184
    RING = 3
    GPC = C // LN
    GPS = S // LN

    @functools.partial(
        pl.kernel,
        mesh=_mesh(),
        out_type=jax.ShapeDtypeStruct((NP, NC * DH), jnp.float32),
        scratch_types=[
            pltpu.VMEM((RING, J, S), jnp.int32),
            pltpu.VMEM((RING, J, S), jnp.int32),
            pltpu.VMEM((RING, J, S), jnp.int32),
            pltpu.VMEM((RING, C), jnp.float32),
            pltpu.VMEM((RING, C, DH), jnp.float32),
            pltpu.VMEM_SHARED((NP, DH), jnp.float32),
            pltpu.SemaphoreType.DMA,
            pltpu.SemaphoreType.DMA,
            pltpu.SemaphoreType.DMA,
        ],
        compiler_params=pltpu.CompilerParams(use_tc_tiling_on_sc=False),
    )
    def spmm(ego2, col2, row2, vals, side, colv, rowL, rowS, valv, rows_v,
             acc, lsem, gsem, ssem):
        c = lax.axis_index("c")
        s = lax.axis_index("s")
        zv = jnp.zeros((LN,), jnp.float32)

        def zfill(i, _):
            rows_v[0, i, pl.ds(0, LN)] = zv
            rows_v[0, i, pl.ds(LN, LN)] = zv
            return _
        lax.fori_loop(0, ZR, zfill, None)
        def zcopy(j, _):
            pltpu.sync_copy(rows_v.at[0, pl.ds(0, ZR)],
                            acc.at[pl.ds(s * RPT + j * ZR, ZR)])
            return _
        lax.fori_loop(0, RPT // ZR, zcopy, None)
        plsc.subcore_barrier()

        rb0 = s * (ET // S)
        eb0 = s * ET

        def issue_linear(k, r):
            pltpu.async_copy(col2.at[pl.ds(rb0 + k * J, J)], colv.at[r], lsem)
            pltpu.async_copy(row2.at[pl.ds(rb0 + k * J, J)], rowL.at[r], lsem)
            pltpu.async_copy(vals.at[pl.ds(eb0 + k * C, C)], valv.at[r], lsem)

        def wait_linear(r):
            pltpu.make_async_copy(col2.at[pl.ds(0, J)], colv.at[r], lsem).wait()
            pltpu.make_async_copy(row2.at[pl.ds(0, J)], rowL.at[r], lsem).wait()
            pltpu.make_async_copy(vals.at[pl.ds(0, C)], valv.at[r], lsem).wait()

        def idx_compute(r):
            def ib(i, _):
                j = i // GPS
                g = i % GPS
                v = colv[r, j, pl.ds(g * LN, LN)]
                colv[r, j, pl.ds(g * LN, LN)] = v * 2 + c
                return _
            lax.fori_loop(0, GPC, ib, None)

        def issue_gather(r):
            for j in range(J):
                pltpu.async_copy(ego2.at[colv.at[r, j]],
                                 rows_v.at[r, pl.ds(j * S, S)], gsem)

        def wait_gather(r):
            for j in range(J):
                pltpu.make_async_copy(ego2.at[colv.at[r, j]],
                                      rows_v.at[r, pl.ds(j * S, S)],
                                      gsem).wait()

        def scale(r):
            def sg(g, _):
                j = g // GPS
                gg = g % GPS
                rowS[r, j, pl.ds(gg * LN, LN)] = rowL[r, j, pl.ds(gg * LN, LN)]
                vv = valv[r, pl.ds(g * LN, LN)]
                for i in range(LN):
                    sval = vv.at[jnp.full((LN,), i, jnp.int32)].get(
                        mode='promise_in_bounds')
                    e = g * LN + i
                    rows_v[r, e, pl.ds(0, LN)] = rows_v[r, e, pl.ds(0, LN)] * sval
                    rows_v[r, e, pl.ds(LN, LN)] = rows_v[r, e, pl.ds(LN, LN)] * sval
                return _
            lax.fori_loop(0, GPC, sg, None)

        def issue_scatter(r):
            for j in range(J):
                pltpu.async_copy(rows_v.at[r, pl.ds(j * S, S)],
                                 acc.at[rowS.at[r, j]], ssem, add=True)

        def wait_scatter(r):
            for j in range(J):
                pltpu.make_async_copy(rows_v.at[r, pl.ds(j * S, S)],
                                      acc.at[rowS.at[r, j]], ssem).wait()

        issue_linear(0, 0)
        issue_linear(1, 1)
        issue_linear(2, 2)
        wait_linear(0)
        idx_compute(0)
        issue_gather(0)

        def step(k, r):
            r1 = (r + 1) % RING

            @pl.when(k + 1 < NCH)
            def _():
                wait_linear(r1)
                idx_compute(r1)

            @pl.when(jnp.logical_and(k + 1 < NCH, k >= 2))
            def _():
                wait_scatter(r1)

            @pl.when(k + 1 < NCH)
            def _():
                issue_gather(r1)

            wait_gather(r)
            scale(r)
            issue_scatter(r)

            @pl.when(k + RING < NCH)
            def _():
                issue_linear(k + RING, r)

        def outer(m, _):
            for u in range(RING):
                step(m * RING + u, u)
            return _
        lax.fori_loop(0, NCH // RING, outer, None)

        for u in range(RING):
            wait_scatter(u)

        plsc.subcore_barrier()
        pltpu.sync_copy(acc.at[pl.ds(s * RPT, RPT)],
                        side.at[pl.ds(s * RPT, RPT), pl.ds(c * DH, DH)])

    return spmm


def _dense_body(side_ref, ego_ref, wg_ref, bg_ref, wb_ref, bb_ref,
                h_ref, n_ref):
    side = side_ref[...]
    ego = ego_ref[...]
    su = jnp.dot(side, wg_ref[...], preferred_element_type=jnp.float32,
                 precision=lax.Precision.HIGHEST)
    bi = jnp.dot(ego * side, wb_ref[...], preferred_element_type=jnp.float32,
                 precision=lax.Precision.HIGHEST)
    x = su + bg_ref[...] + bi + bb_ref[...]
    h = jnp.where(x >= 0, x, 0.2 * x)
    h_ref[...] = h
    nrm = jnp.sqrt(jnp.sum(h * h, axis=1, keepdims=True))
    n_ref[...] = h / jnp.maximum(nrm, 1e-12)


def _dense(ego, side, wg, bg, wb, bb, blk=2176):
    N, D = ego.shape
    grid = N // blk
    return pl.pallas_call(
        _dense_body,
        grid=(grid,),
        in_specs=[
            pl.BlockSpec((blk, D), lambda i: (i, 0)),
            pl.BlockSpec((blk, D), lambda i: (i, 0)),
            pl.BlockSpec((D, D), lambda i: (0, 0)),
            pl.BlockSpec((1, D), lambda i: (0, 0)),
            pl.BlockSpec((D, D), lambda i: (0, 0)),
            pl.BlockSpec((1, D), lambda i: (0, 0)),
        ],
        out_specs=[pl.BlockSpec((blk, D), lambda i: (i, 0)),
                   pl.BlockSpec((blk, D), lambda i: (i, 0))],
        out_shape=[jax.ShapeDtypeStruct((N, D), jnp.float32),
                   jax.ShapeDtypeStruct((N, D), jnp.float32)],
    )(side, ego, wg, bg, wb, bb)


def _make_gather_out(N_user, B, D):
    BPW = B // (NC * NS)

    @functools.partial(
        pl.kernel,
        mesh=_mesh(),
        out_type=jax.ShapeDtypeStruct((2, 3, B, D), jnp.float32),
        scratch_types=[
            pltpu.VMEM((BPW,), jnp.int32),
            pltpu.VMEM((BPW, D), jnp.float32),
            pltpu.SemaphoreType.DMA,
        ],
        compiler_params=pltpu.CompilerParams(use_tc_tiling_on_sc=False),
    )
    def gather_out(ego0, norm1, norm2, users, items, out, idxv, rows_v, sem):
        c = lax.axis_index("c")
        s = lax.axis_index("s")
        w = s * NC + c
        tabs = [ego0, norm1, norm2]
        for ui, src in enumerate([users, items]):
            pltpu.sync_copy(src.at[pl.ds(w * BPW, BPW)], idxv)
            if ui == 1:
                def off(g, _):
                    idxv[pl.ds(g * LN, LN)] = idxv[pl.ds(g * LN, LN)] + N_user
                    return _
                lax.fori_loop(0, BPW // LN, off, None)
            for t in range(3):
                pltpu.async_copy(tabs[t].at[idxv], rows_v, sem).wait()
                pltpu.sync_copy(rows_v, out.at[ui, t, pl.ds(w * BPW, BPW)])

    return gather_out


def kernel(users, items, adj_indices, adj_vals, user_emb, item_emb,
           W_gc_0, b_gc_0, W_bi_0, b_bi_0, W_gc_1, b_gc_1, W_bi_1, b_bi_1):
    N_user, D = user_emb.shape
    N = N_user + item_emb.shape[0]
    E = adj_vals.shape[0]
    B = users.shape[0]

    C, S = 256, 128
    EQ = NS * C * 3
    E_pad = ((E + EQ - 1) // EQ) * EQ
    pad = E_pad - E
    NP = ((N + NS * 8 - 1) // (NS * 8)) * (NS * 8)

    row = adj_indices[0]
    col = adj_indices[1]
    if pad:
        z = jnp.zeros((pad,), jnp.int32)
        row = jnp.concatenate([row, z])
        col = jnp.concatenate([col, z])
        vals = jnp.concatenate([adj_vals, jnp.zeros((pad,), jnp.float32)])
    else:
        vals = adj_vals
    col2 = col.reshape(E_pad // S, S)
    row2 = row.reshape(E_pad // S, S)

    spmm = _make_spmm(NP, E_pad, C, S)
    gather_out = _make_gather_out(N_user, B, D)

    ego0 = jnp.concatenate(
        [user_emb, item_emb,
         jnp.zeros((NP - N, D), jnp.float32)], axis=0)

    side1 = spmm(ego0.reshape(2 * NP, DH), col2, row2, vals)
    h1, norm1 = _dense(ego0, side1, W_gc_0, b_gc_0, W_bi_0, b_bi_0)

    side2 = spmm(h1.reshape(2 * NP, DH), col2, row2, vals)
    h2, norm2 = _dense(h1, side2, W_gc_1, b_gc_1, W_bi_1, b_bi_1)

    outg = gather_out(ego0, norm1, norm2, users, items)
    u_g = outg[0].transpose(1, 0, 2).reshape(B, 3 * D)
    i_g = outg[1].transpose(1, 0, 2).reshape(B, 3 * D)
    return (u_g, i_g)
